# baseline (speedup 1.0000x reference)
import jax
import jax.numpy as jnp
from jax import lax
from jax.experimental import pallas as pl
from jax.experimental.pallas import tpu as pltpu, tpu_sc as plsc

N_STATIONS = 256
N_FORECAST = 730
N_STEPS = 81
F_DIM = 4
N_CELLS = N_STATIONS * N_FORECAST * N_STEPS

NC = 2
NS = 16
NW = NC * NS
LANES = 16


def _emos_body(wtab, btab, featT, sid, fid, tid, out_hbm,
               sid_v, fid_v, tid_v, feat_v, idxw, idxb, wc, brows, out_v,
               semw, semb):
    spt = out_v.shape[0]
    n_groups = spt // LANES
    blk = spt // 128
    n_chunks = 9 * blk

    wid = lax.axis_index("s") * NC + lax.axis_index("c")
    base = wid * spt

    pltpu.sync_copy(sid.at[pl.ds(base, spt)], sid_v)
    pltpu.sync_copy(fid.at[pl.ds(base, spt)], fid_v)
    pltpu.sync_copy(tid.at[pl.ds(base, spt)], tid_v)
    for c in range(F_DIM):
        pltpu.sync_copy(featT.at[c, pl.ds(base, spt)], feat_v.at[c])

    def idx_body(g, carry):
        off = g * LANES
        s = sid_v[pl.ds(off, LANES)]
        f = fid_v[pl.ds(off, LANES)]
        t = tid_v[pl.ds(off, LANES)]
        sb = s * (N_FORECAST * N_STEPS)
        f0 = jnp.where(f == 0, N_FORECAST - 1, f - 1) * N_STEPS
        f1 = f * N_STEPS
        f2 = jnp.where(f == N_FORECAST - 1, 0, f + 1) * N_STEPS
        t0 = jnp.where(t == 0, t, t - 1)
        t2 = jnp.where(t == N_STEPS - 1, t, t + 1)
        row0 = g // 8
        col = (g % 8) * LANES
        m = 0
        for fr in (f0, f1, f2):
            for tc in (t0, t, t2):
                cell = sb + fr + tc
                row = m * blk + row0
                idxb[row, pl.ds(col, LANES)] = cell
                w0 = cell * F_DIM
                for c in range(F_DIM):
                    idxw[c, row, pl.ds(col, LANES)] = w0 + c
                m += 1
        return carry

    lax.fori_loop(0, n_groups, idx_body, 0)

    def gather_body(r, carry):
        cps = [pltpu.make_async_copy(wtab.at[idxw.at[c, r]], wc.at[c, r], semw)
               for c in range(F_DIM)]
        cpb = pltpu.make_async_copy(btab.at[idxb.at[r]], brows.at[r], semb)
        for cp in cps:
            cp.start()
        cpb.start()
        for cp in cps:
            cp.wait()
        cpb.wait()
        return carry

    lax.fori_loop(0, n_chunks, gather_body, 0)

    def comp_body(g, carry):
        i0 = g * LANES
        row0 = g // 8
        col = (g % 8) * LANES
        pred = jnp.zeros(LANES, jnp.float32)
        for c in range(F_DIM):
            acc = jnp.zeros(LANES, jnp.float32)
            for m in range(9):
                acc = acc + wc[c, m * blk + row0, pl.ds(col, LANES)]
            pred = pred + acc * feat_v[c, pl.ds(i0, LANES)]
        accb = jnp.zeros(LANES, jnp.float32)
        for m in range(9):
            accb = accb + brows[m * blk + row0, pl.ds(col, LANES)]
        out_v[pl.ds(i0, LANES)] = (pred + accb) * jnp.float32(1.0 / 9.0)
        return carry

    lax.fori_loop(0, n_groups, comp_body, 0)

    pltpu.sync_copy(out_v, out_hbm.at[pl.ds(base, spt)])


def kernel(features, station_id, forecast_id, step_id, weights, biases):
    b = features.shape[0]
    spt = b // NW
    n_chunks = (9 * spt) // 128
    wtab = weights.reshape(N_CELLS * F_DIM)
    btab = biases.reshape(N_CELLS)
    featT = features.T

    mesh = plsc.VectorSubcoreMesh(core_axis_name="c", subcore_axis_name="s")
    run = pl.kernel(
        _emos_body,
        out_type=jax.ShapeDtypeStruct((b,), jnp.float32),
        mesh=mesh,
        scratch_types=[
            pltpu.VMEM((spt,), jnp.int32),
            pltpu.VMEM((spt,), jnp.int32),
            pltpu.VMEM((spt,), jnp.int32),
            pltpu.VMEM((F_DIM, spt), jnp.float32),
            pltpu.VMEM((F_DIM, n_chunks, 128), jnp.int32),
            pltpu.VMEM((n_chunks, 128), jnp.int32),
            pltpu.VMEM((F_DIM, n_chunks, 128), jnp.float32),
            pltpu.VMEM((n_chunks, 128), jnp.float32),
            pltpu.VMEM((spt,), jnp.float32),
            pltpu.SemaphoreType.DMA,
            pltpu.SemaphoreType.DMA,
        ],
    )
    return run(wtab, btab, featT,
               station_id.astype(jnp.int32),
               forecast_id.astype(jnp.int32),
               step_id.astype(jnp.int32))


# speedup vs baseline: 59.6706x; 59.6706x over previous
import jax
import jax.numpy as jnp
from jax import lax
from jax.experimental import pallas as pl
from jax.experimental.pallas import tpu as pltpu, tpu_sc as plsc

N_STATIONS = 256
N_FORECAST = 730
N_STEPS = 81
F_DIM = 4
N_CELLS = N_STATIONS * N_FORECAST * N_STEPS

NC = 2
NS = 16
NW = NC * NS
LANES = 16


def _emos_body(wtab, btab, featT, sid, fid, tid, out_hbm,
               sid_v, fid_v, tid_v, feat_v, idxw, idxb, wc, brows, out_v,
               semw, semb):
    spt = out_v.shape[0]
    n_groups = spt // LANES
    blk = spt // 128
    n_chunks = 9 * blk

    wid = lax.axis_index("s") * NC + lax.axis_index("c")
    base = wid * spt

    pltpu.sync_copy(sid.at[pl.ds(base, spt)], sid_v)
    pltpu.sync_copy(fid.at[pl.ds(base, spt)], fid_v)
    pltpu.sync_copy(tid.at[pl.ds(base, spt)], tid_v)
    for c in range(F_DIM):
        pltpu.sync_copy(featT.at[c, pl.ds(base, spt)], feat_v.at[c])

    def idx_body(g, carry):
        off = g * LANES
        s = sid_v[pl.ds(off, LANES)]
        f = fid_v[pl.ds(off, LANES)]
        t = tid_v[pl.ds(off, LANES)]
        f0 = jnp.where(f == 0, N_FORECAST - 1, f - 1) * N_STEPS
        f1 = f * N_STEPS
        f2 = jnp.where(f == N_FORECAST - 1, 0, f + 1) * N_STEPS
        t0 = jnp.where(t == 0, t, t - 1)
        t2 = jnp.where(t == N_STEPS - 1, t, t + 1)
        sw = lax.shift_right_logical(s, 7) * 512 + lax.bitwise_and(s, 127)
        row0 = g // 8
        col = (g % 8) * LANES
        m = 0
        for fr in (f0, f1, f2):
            for tc in (t0, t, t2):
                ft = fr + tc
                row = m * blk + row0
                idxb[row, pl.ds(col, LANES)] = ft * 256 + s
                w0 = ft * 1024 + sw
                for c in range(F_DIM):
                    idxw[c, row, pl.ds(col, LANES)] = w0 + c * 128
                m += 1
        return carry

    lax.fori_loop(0, n_groups, idx_body, 0)

    def gather_body(r, carry):
        cps = [pltpu.make_async_copy(wtab.at[idxw.at[c, r]], wc.at[c, r], semw)
               for c in range(F_DIM)]
        cpb = pltpu.make_async_copy(btab.at[idxb.at[r]], brows.at[r], semb)
        for cp in cps:
            cp.start()
        cpb.start()
        for cp in cps:
            cp.wait()
        cpb.wait()
        return carry

    lax.fori_loop(0, n_chunks, gather_body, 0)

    def comp_body(g, carry):
        i0 = g * LANES
        row0 = g // 8
        col = (g % 8) * LANES
        pred = jnp.zeros(LANES, jnp.float32)
        for c in range(F_DIM):
            acc = jnp.zeros(LANES, jnp.float32)
            for m in range(9):
                acc = acc + wc[c, m * blk + row0, pl.ds(col, LANES)]
            pred = pred + acc * feat_v[c, pl.ds(i0, LANES)]
        accb = jnp.zeros(LANES, jnp.float32)
        for m in range(9):
            accb = accb + brows[m * blk + row0, pl.ds(col, LANES)]
        out_v[pl.ds(i0, LANES)] = (pred + accb) * jnp.float32(1.0 / 9.0)
        return carry

    lax.fori_loop(0, n_groups, comp_body, 0)

    pltpu.sync_copy(out_v, out_hbm.at[pl.ds(base, spt)])


def kernel(features, station_id, forecast_id, step_id, weights, biases):
    b = features.shape[0]
    spt = b // NW
    n_chunks = (9 * spt) // 128
    nft = N_FORECAST * N_STEPS
    wtab = (weights.transpose(1, 2, 3, 0)
            .reshape(nft, F_DIM, N_STATIONS // 128, 128)
            .transpose(0, 2, 1, 3)
            .reshape(N_CELLS * F_DIM))
    btab = biases[:, :, :, 0].transpose(1, 2, 0).reshape(N_CELLS)
    featT = features.T

    mesh = plsc.VectorSubcoreMesh(core_axis_name="c", subcore_axis_name="s")
    run = pl.kernel(
        _emos_body,
        out_type=jax.ShapeDtypeStruct((b,), jnp.float32),
        mesh=mesh,
        scratch_types=[
            pltpu.VMEM((spt,), jnp.int32),
            pltpu.VMEM((spt,), jnp.int32),
            pltpu.VMEM((spt,), jnp.int32),
            pltpu.VMEM((F_DIM, spt), jnp.float32),
            pltpu.VMEM((F_DIM, n_chunks, 128), jnp.int32),
            pltpu.VMEM((n_chunks, 128), jnp.int32),
            pltpu.VMEM((F_DIM, n_chunks, 128), jnp.float32),
            pltpu.VMEM((n_chunks, 128), jnp.float32),
            pltpu.VMEM((spt,), jnp.float32),
            pltpu.SemaphoreType.DMA,
            pltpu.SemaphoreType.DMA,
        ],
    )
    return run(wtab, btab, featT,
               station_id.astype(jnp.int32),
               forecast_id.astype(jnp.int32),
               step_id.astype(jnp.int32))


# speedup vs baseline: 145.4472x; 2.4375x over previous
import jax
import jax.numpy as jnp
from jax import lax
from jax.experimental import pallas as pl
from jax.experimental.pallas import tpu as pltpu, tpu_sc as plsc

N_STATIONS = 256
N_FORECAST = 730
N_STEPS = 81
F_DIM = 4
N_CELLS = N_STATIONS * N_FORECAST * N_STEPS

NC = 2
NS = 16
NW = NC * NS
LANES = 16


def _emos_body(wtab, btab, featT, sid, fid, tid, out_hbm,
               sid_v, fid_v, tid_v, feat_v, idxw, idxb, wc, brows, out_v,
               semw, semb):
    spt = out_v.shape[0]
    n_groups = spt // LANES
    blk = spt // 128
    n_chunks = 9 * blk

    wid = lax.axis_index("s") * NC + lax.axis_index("c")
    base = wid * spt

    pltpu.sync_copy(sid.at[pl.ds(base, spt)], sid_v)
    pltpu.sync_copy(fid.at[pl.ds(base, spt)], fid_v)
    pltpu.sync_copy(tid.at[pl.ds(base, spt)], tid_v)
    for c in range(F_DIM):
        pltpu.sync_copy(featT.at[c, pl.ds(base, spt)], feat_v.at[c])

    def idx_body(g, carry):
        off = g * LANES
        s = sid_v[pl.ds(off, LANES)]
        f = fid_v[pl.ds(off, LANES)]
        t = tid_v[pl.ds(off, LANES)]
        f0 = jnp.where(f == 0, N_FORECAST - 1, f - 1) * N_STEPS
        f1 = f * N_STEPS
        f2 = jnp.where(f == N_FORECAST - 1, 0, f + 1) * N_STEPS
        t0 = jnp.where(t == 0, t, t - 1)
        t2 = jnp.where(t == N_STEPS - 1, t, t + 1)
        sw = lax.shift_right_logical(s, 7) * 512 + lax.bitwise_and(s, 127)
        row0 = g // 8
        col = (g % 8) * LANES
        m = 0
        for fr in (f0, f1, f2):
            for tc in (t0, t, t2):
                ft = fr + tc
                row = m * blk + row0
                idxb[row, pl.ds(col, LANES)] = ft * 256 + s
                w0 = ft * 1024 + sw
                for c in range(F_DIM):
                    idxw[c, row, pl.ds(col, LANES)] = w0 + c * 128
                m += 1
        return carry

    lax.fori_loop(0, n_groups, idx_body, 0)

    def gather_body(r, carry):
        cps = [pltpu.make_async_copy(wtab.at[idxw.at[c, r]], wc.at[c, r], semw)
               for c in range(F_DIM)]
        cpb = pltpu.make_async_copy(btab.at[idxb.at[r]], brows.at[r], semb)
        for cp in cps:
            cp.start()
        cpb.start()
        for cp in cps:
            cp.wait()
        cpb.wait()
        return carry

    lax.fori_loop(0, n_chunks, gather_body, 0)

    def comp_body(g, carry):
        i0 = g * LANES
        row0 = g // 8
        col = (g % 8) * LANES
        pred = jnp.zeros(LANES, jnp.float32)
        for c in range(F_DIM):
            acc = jnp.zeros(LANES, jnp.float32)
            for m in range(9):
                acc = acc + wc[c, m * blk + row0, pl.ds(col, LANES)]
            pred = pred + acc * feat_v[c, pl.ds(i0, LANES)]
        accb = jnp.zeros(LANES, jnp.float32)
        for m in range(9):
            accb = accb + brows[m * blk + row0, pl.ds(col, LANES)]
        out_v[pl.ds(i0, LANES)] = (pred + accb) * jnp.float32(1.0 / 9.0)
        return carry

    lax.fori_loop(0, n_groups, comp_body, 0)

    pltpu.sync_copy(out_v, out_hbm.at[pl.ds(base, spt)])


def kernel(features, station_id, forecast_id, step_id, weights, biases):
    b = features.shape[0]
    spt = b // NW
    n_chunks = (9 * spt) // 128
    nft = N_FORECAST * N_STEPS
    wtab = (weights.transpose(1, 2, 3, 0)
            .reshape(nft, F_DIM, N_STATIONS // 128, 128)
            .transpose(0, 2, 1, 3)
            .reshape(N_CELLS * F_DIM))
    btab = (biases.reshape(N_STATIONS // 128, 128, nft)
            .transpose(2, 0, 1)
            .reshape(N_CELLS))
    featT = features.T

    mesh = plsc.VectorSubcoreMesh(core_axis_name="c", subcore_axis_name="s")
    run = pl.kernel(
        _emos_body,
        out_type=jax.ShapeDtypeStruct((b,), jnp.float32),
        mesh=mesh,
        scratch_types=[
            pltpu.VMEM((spt,), jnp.int32),
            pltpu.VMEM((spt,), jnp.int32),
            pltpu.VMEM((spt,), jnp.int32),
            pltpu.VMEM((F_DIM, spt), jnp.float32),
            pltpu.VMEM((F_DIM, n_chunks, 128), jnp.int32),
            pltpu.VMEM((n_chunks, 128), jnp.int32),
            pltpu.VMEM((F_DIM, n_chunks, 128), jnp.float32),
            pltpu.VMEM((n_chunks, 128), jnp.float32),
            pltpu.VMEM((spt,), jnp.float32),
            pltpu.SemaphoreType.DMA,
            pltpu.SemaphoreType.DMA,
        ],
    )
    return run(wtab, btab, featT,
               station_id.astype(jnp.int32),
               forecast_id.astype(jnp.int32),
               step_id.astype(jnp.int32))


# speedup vs baseline: 159.7418x; 1.0983x over previous
import jax
import jax.numpy as jnp
from jax import lax
from jax.experimental import pallas as pl
from jax.experimental.pallas import tpu as pltpu, tpu_sc as plsc

N_STATIONS = 256
N_FORECAST = 730
N_STEPS = 81
F_DIM = 4
N_CELLS = N_STATIONS * N_FORECAST * N_STEPS

NC = 2
NS = 16
NW = NC * NS
LANES = 16


def _emos_body(wtab, btab, featT, sid, fid, tid, out_hbm,
               sid_v, fid_v, tid_v, feat_v, idxw, idxb, wc, brows, out_v,
               semw, semb):
    spt = out_v.shape[0]
    n_groups = spt // LANES
    blk = spt // 128
    n_chunks = 9 * blk

    wid = lax.axis_index("s") * NC + lax.axis_index("c")
    base = wid * spt

    pltpu.sync_copy(sid.at[pl.ds(base, spt)], sid_v)
    pltpu.sync_copy(fid.at[pl.ds(base, spt)], fid_v)
    pltpu.sync_copy(tid.at[pl.ds(base, spt)], tid_v)
    for c in range(F_DIM):
        pltpu.sync_copy(featT.at[c, pl.ds(base, spt)], feat_v.at[c])

    def idx_body(g, carry):
        off = g * LANES
        s = sid_v[pl.ds(off, LANES)]
        f = fid_v[pl.ds(off, LANES)]
        t = tid_v[pl.ds(off, LANES)]
        f0 = jnp.where(f == 0, N_FORECAST - 1, f - 1) * N_STEPS
        f1 = f * N_STEPS
        f2 = jnp.where(f == N_FORECAST - 1, 0, f + 1) * N_STEPS
        t0 = jnp.where(t == 0, t, t - 1)
        t2 = jnp.where(t == N_STEPS - 1, t, t + 1)
        sw = lax.shift_right_logical(s, 7) * 512 + lax.bitwise_and(s, 127)
        row0 = g // 8
        col = (g % 8) * LANES
        m = 0
        for fr in (f0, f1, f2):
            for tc in (t0, t, t2):
                ft = fr + tc
                row = m * blk + row0
                idxb[row, pl.ds(col, LANES)] = ft * 256 + s
                w0 = ft * 1024 + sw
                for c in range(F_DIM):
                    idxw[c, row, pl.ds(col, LANES)] = w0 + c * 128
                m += 1
        return carry

    lax.fori_loop(0, n_groups, idx_body, 0)

    def gather_start(r):
        for c in range(F_DIM):
            pltpu.make_async_copy(wtab.at[idxw.at[c, r]], wc.at[c, r], semw).start()
        pltpu.make_async_copy(btab.at[idxb.at[r]], brows.at[r], semb).start()

    def gather_wait(r):
        for c in range(F_DIM):
            pltpu.make_async_copy(wtab.at[idxw.at[c, r]], wc.at[c, r], semw).wait()
        pltpu.make_async_copy(btab.at[idxb.at[r]], brows.at[r], semb).wait()

    gather_start(0)
    gather_start(1)

    def gather_body(r, carry):
        gather_start(r + 2)
        gather_wait(r)
        return carry

    lax.fori_loop(0, n_chunks - 2, gather_body, 0)
    gather_wait(n_chunks - 2)
    gather_wait(n_chunks - 1)

    def comp_body(g, carry):
        i0 = g * LANES
        row0 = g // 8
        col = (g % 8) * LANES
        pred = jnp.zeros(LANES, jnp.float32)
        for c in range(F_DIM):
            acc = jnp.zeros(LANES, jnp.float32)
            for m in range(9):
                acc = acc + wc[c, m * blk + row0, pl.ds(col, LANES)]
            pred = pred + acc * feat_v[c, pl.ds(i0, LANES)]
        accb = jnp.zeros(LANES, jnp.float32)
        for m in range(9):
            accb = accb + brows[m * blk + row0, pl.ds(col, LANES)]
        out_v[pl.ds(i0, LANES)] = (pred + accb) * jnp.float32(1.0 / 9.0)
        return carry

    lax.fori_loop(0, n_groups, comp_body, 0)

    pltpu.sync_copy(out_v, out_hbm.at[pl.ds(base, spt)])


def kernel(features, station_id, forecast_id, step_id, weights, biases):
    b = features.shape[0]
    spt = b // NW
    n_chunks = (9 * spt) // 128
    nft = N_FORECAST * N_STEPS
    wtab = (weights.transpose(1, 2, 3, 0)
            .reshape(nft, F_DIM, N_STATIONS // 128, 128)
            .transpose(0, 2, 1, 3)
            .reshape(N_CELLS * F_DIM))
    btab = (biases.reshape(N_STATIONS // 128, 128, nft)
            .transpose(2, 0, 1)
            .reshape(N_CELLS))
    featT = features.T

    mesh = plsc.VectorSubcoreMesh(core_axis_name="c", subcore_axis_name="s")
    run = pl.kernel(
        _emos_body,
        out_type=jax.ShapeDtypeStruct((b,), jnp.float32),
        mesh=mesh,
        scratch_types=[
            pltpu.VMEM((spt,), jnp.int32),
            pltpu.VMEM((spt,), jnp.int32),
            pltpu.VMEM((spt,), jnp.int32),
            pltpu.VMEM((F_DIM, spt), jnp.float32),
            pltpu.VMEM((F_DIM, n_chunks, 128), jnp.int32),
            pltpu.VMEM((n_chunks, 128), jnp.int32),
            pltpu.VMEM((F_DIM, n_chunks, 128), jnp.float32),
            pltpu.VMEM((n_chunks, 128), jnp.float32),
            pltpu.VMEM((spt,), jnp.float32),
            pltpu.SemaphoreType.DMA,
            pltpu.SemaphoreType.DMA,
        ],
    )
    return run(wtab, btab, featT,
               station_id.astype(jnp.int32),
               forecast_id.astype(jnp.int32),
               step_id.astype(jnp.int32))


# speedup vs baseline: 352.8839x; 2.2091x over previous
import jax
import jax.numpy as jnp
from jax import lax
from jax.experimental import pallas as pl
from jax.experimental.pallas import tpu as pltpu, tpu_sc as plsc

N_STATIONS = 256
N_FORECAST = 730
N_STEPS = 81
F_DIM = 4
N_CELLS = N_STATIONS * N_FORECAST * N_STEPS

NC = 2
NS = 16
NW = NC * NS
LANES = 16


def _emos_body(wtab, btab, featT, sid, fid, tid, out_hbm,
               sid_v, fid_v, tid_v, feat_v, idxw, idxb, wc, brows, out_v,
               semw, semb):
    spt = out_v.shape[0]
    n_groups = spt // LANES
    blk = spt // 128
    n_chunks = 9 * blk

    wid = lax.axis_index("s") * NC + lax.axis_index("c")
    base = wid * spt

    pltpu.sync_copy(sid.at[pl.ds(base, spt)], sid_v)
    pltpu.sync_copy(fid.at[pl.ds(base, spt)], fid_v)
    pltpu.sync_copy(tid.at[pl.ds(base, spt)], tid_v)
    for c in range(F_DIM):
        pltpu.sync_copy(featT.at[c, pl.ds(base, spt)], feat_v.at[c])

    def idx_body(g, carry):
        off = g * LANES
        s = sid_v[pl.ds(off, LANES)]
        f = fid_v[pl.ds(off, LANES)]
        t = tid_v[pl.ds(off, LANES)]
        f0 = jnp.where(f == 0, N_FORECAST - 1, f - 1) * N_STEPS
        f1 = f * N_STEPS
        f2 = jnp.where(f == N_FORECAST - 1, 0, f + 1) * N_STEPS
        t0 = jnp.where(t == 0, t, t - 1)
        t2 = jnp.where(t == N_STEPS - 1, t, t + 1)
        sw = lax.shift_right_logical(s, 7) * 512 + lax.bitwise_and(s, 127)
        row0 = g // 8
        col = (g % 8) * LANES
        m = 0
        for fr in (f0, f1, f2):
            for tc in (t0, t, t2):
                ft = fr + tc
                row = m * blk + row0
                idxb[row, pl.ds(col, LANES)] = ft * 256 + s
                w0 = ft * 1024 + sw
                for c in range(F_DIM):
                    idxw[c, row, pl.ds(col, LANES)] = w0 + c * 128
                m += 1
        return carry

    lax.fori_loop(0, n_groups, idx_body, 0)

    def gather_start(r):
        for c in range(F_DIM):
            pltpu.make_async_copy(wtab.at[idxw.at[c, r]], wc.at[c, r], semw).start()
        pltpu.make_async_copy(btab.at[0].at[idxb.at[r]], brows.at[r], semb).start()

    def gather_wait(r):
        for c in range(F_DIM):
            pltpu.make_async_copy(wtab.at[idxw.at[c, r]], wc.at[c, r], semw).wait()
        pltpu.make_async_copy(btab.at[0].at[idxb.at[r]], brows.at[r], semb).wait()

    gather_start(0)
    gather_start(1)

    def gather_body(r, carry):
        gather_start(r + 2)
        gather_wait(r)
        return carry

    lax.fori_loop(0, n_chunks - 2, gather_body, 0)
    gather_wait(n_chunks - 2)
    gather_wait(n_chunks - 1)

    def comp_body(g, carry):
        i0 = g * LANES
        row0 = g // 8
        col = (g % 8) * LANES
        pred = jnp.zeros(LANES, jnp.float32)
        for c in range(F_DIM):
            acc = jnp.zeros(LANES, jnp.float32)
            for m in range(9):
                acc = acc + wc[c, m * blk + row0, pl.ds(col, LANES)]
            pred = pred + acc * feat_v[c, pl.ds(i0, LANES)]
        accb = jnp.zeros(LANES, jnp.float32)
        for m in range(9):
            accb = accb + brows[m * blk + row0, pl.ds(col, LANES)]
        out_v[pl.ds(i0, LANES)] = (pred + accb) * jnp.float32(1.0 / 9.0)
        return carry

    lax.fori_loop(0, n_groups, comp_body, 0)

    pltpu.sync_copy(out_v, out_hbm.at[pl.ds(base, spt)])


def kernel(features, station_id, forecast_id, step_id, weights, biases):
    b = features.shape[0]
    spt = b // NW
    n_chunks = (9 * spt) // 128
    nft = N_FORECAST * N_STEPS
    wtab = (weights.transpose(1, 2, 3, 0)
            .reshape(nft, F_DIM, N_STATIONS // 128, 128)
            .transpose(0, 2, 1, 3)
            .reshape(N_CELLS * F_DIM))
    btab = (biases.reshape(N_STATIONS // 128, 128, nft)
            .transpose(2, 0, 1)
            .reshape(1, N_CELLS))
    featT = features.T

    mesh = plsc.VectorSubcoreMesh(core_axis_name="c", subcore_axis_name="s")
    run = pl.kernel(
        _emos_body,
        out_type=jax.ShapeDtypeStruct((b,), jnp.float32),
        mesh=mesh,
        scratch_types=[
            pltpu.VMEM((spt,), jnp.int32),
            pltpu.VMEM((spt,), jnp.int32),
            pltpu.VMEM((spt,), jnp.int32),
            pltpu.VMEM((F_DIM, spt), jnp.float32),
            pltpu.VMEM((F_DIM, n_chunks, 128), jnp.int32),
            pltpu.VMEM((n_chunks, 128), jnp.int32),
            pltpu.VMEM((F_DIM, n_chunks, 128), jnp.float32),
            pltpu.VMEM((n_chunks, 128), jnp.float32),
            pltpu.VMEM((spt,), jnp.float32),
            pltpu.SemaphoreType.DMA,
            pltpu.SemaphoreType.DMA,
        ],
    )
    return run(wtab, btab, featT,
               station_id.astype(jnp.int32),
               forecast_id.astype(jnp.int32),
               step_id.astype(jnp.int32))


# speedup vs baseline: 369.3584x; 1.0467x over previous
import jax
import jax.numpy as jnp
from jax import lax
from jax.experimental import pallas as pl
from jax.experimental.pallas import tpu as pltpu, tpu_sc as plsc

N_STATIONS = 256
N_FORECAST = 730
N_STEPS = 81
F_DIM = 4
N_CELLS = N_STATIONS * N_FORECAST * N_STEPS

NC = 2
NS = 16
NW = NC * NS
LANES = 16


def _emos_body(wtab, btab, featT, sid, fid, tid, out_hbm,
               sid_v, fid_v, tid_v, feat_v, idxw, idxb, wc, brows, out_v,
               semw, semb):
    spt = out_v.shape[0]
    n_groups = spt // LANES
    blk = spt // 128
    n_chunks = 9 * blk

    wid = lax.axis_index("s") * NC + lax.axis_index("c")
    base = wid * spt

    pltpu.sync_copy(sid.at[pl.ds(base, spt)], sid_v)
    pltpu.sync_copy(fid.at[pl.ds(base, spt)], fid_v)
    pltpu.sync_copy(tid.at[pl.ds(base, spt)], tid_v)
    for c in range(F_DIM):
        pltpu.sync_copy(featT.at[c, pl.ds(base, spt)], feat_v.at[c])

    def idx_body(g, carry):
        off = g * LANES
        s = sid_v[pl.ds(off, LANES)]
        f = fid_v[pl.ds(off, LANES)]
        t = tid_v[pl.ds(off, LANES)]
        f0 = jnp.where(f == 0, N_FORECAST - 1, f - 1) * N_STEPS
        f1 = f * N_STEPS
        f2 = jnp.where(f == N_FORECAST - 1, 0, f + 1) * N_STEPS
        t0 = jnp.where(t == 0, t, t - 1)
        t2 = jnp.where(t == N_STEPS - 1, t, t + 1)
        sw = lax.shift_right_logical(s, 7) * 512 + lax.bitwise_and(s, 127)
        row0 = g // 8
        col = (g % 8) * LANES
        m = 0
        for fr in (f0, f1, f2):
            for tc in (t0, t, t2):
                ft = fr + tc
                row = m * blk + row0
                idxb[row, pl.ds(col, LANES)] = ft * 256 + s
                w0 = ft * 1024 + sw
                for c in range(F_DIM):
                    idxw[c, row, pl.ds(col, LANES)] = w0 + c * 128
                m += 1
        return carry

    lax.fori_loop(0, n_groups, idx_body, 0)

    def gather_start(r):
        for c in range(F_DIM):
            pltpu.make_async_copy(wtab.at[idxw.at[c, r]], wc.at[c, r], semw).start()
        pltpu.make_async_copy(btab.at[0].at[idxb.at[r]], brows.at[r], semb).start()

    def gather_wait(r):
        for c in range(F_DIM):
            pltpu.make_async_copy(wtab.at[idxw.at[c, r]], wc.at[c, r], semw).wait()
        pltpu.make_async_copy(btab.at[0].at[idxb.at[r]], brows.at[r], semb).wait()

    depth = 4
    for r in range(depth):
        gather_start(r)

    def gather_body(r, carry):
        gather_start(r + depth)
        gather_wait(r)
        return carry

    lax.fori_loop(0, n_chunks - depth, gather_body, 0)
    for r in range(n_chunks - depth, n_chunks):
        gather_wait(r)

    def comp_body(g, carry):
        i0 = g * LANES
        row0 = g // 8
        col = (g % 8) * LANES
        pred = jnp.zeros(LANES, jnp.float32)
        for c in range(F_DIM):
            acc = jnp.zeros(LANES, jnp.float32)
            for m in range(9):
                acc = acc + wc[c, m * blk + row0, pl.ds(col, LANES)]
            pred = pred + acc * feat_v[c, pl.ds(i0, LANES)]
        accb = jnp.zeros(LANES, jnp.float32)
        for m in range(9):
            accb = accb + brows[m * blk + row0, pl.ds(col, LANES)]
        out_v[pl.ds(i0, LANES)] = (pred + accb) * jnp.float32(1.0 / 9.0)
        return carry

    lax.fori_loop(0, n_groups, comp_body, 0)

    pltpu.sync_copy(out_v, out_hbm.at[pl.ds(base, spt)])


def kernel(features, station_id, forecast_id, step_id, weights, biases):
    b = features.shape[0]
    spt = b // NW
    n_chunks = (9 * spt) // 128
    nft = N_FORECAST * N_STEPS
    wtab = (weights.transpose(1, 2, 3, 0)
            .reshape(nft, F_DIM, N_STATIONS // 128, 128)
            .transpose(0, 2, 1, 3)
            .reshape(N_CELLS * F_DIM))
    btab = (biases.reshape(N_STATIONS // 128, 128, nft)
            .transpose(2, 0, 1)
            .reshape(1, N_CELLS))
    featT = features.T

    mesh = plsc.VectorSubcoreMesh(core_axis_name="c", subcore_axis_name="s",
                              num_cores=NC, num_subcores=NS)
    run = pl.kernel(
        _emos_body,
        out_type=jax.ShapeDtypeStruct((b,), jnp.float32),
        mesh=mesh,
        scratch_types=[
            pltpu.VMEM((spt,), jnp.int32),
            pltpu.VMEM((spt,), jnp.int32),
            pltpu.VMEM((spt,), jnp.int32),
            pltpu.VMEM((F_DIM, spt), jnp.float32),
            pltpu.VMEM((F_DIM, n_chunks, 128), jnp.int32),
            pltpu.VMEM((n_chunks, 128), jnp.int32),
            pltpu.VMEM((F_DIM, n_chunks, 128), jnp.float32),
            pltpu.VMEM((n_chunks, 128), jnp.float32),
            pltpu.VMEM((spt,), jnp.float32),
            pltpu.SemaphoreType.DMA,
            pltpu.SemaphoreType.DMA,
        ],
    )
    return run(wtab, btab, featT,
               station_id.astype(jnp.int32),
               forecast_id.astype(jnp.int32),
               step_id.astype(jnp.int32))
